# baseline (speedup 1.0000x reference)
import functools

import jax
import jax.numpy as jnp
from jax import lax
from jax.experimental import pallas as pl
from jax.experimental.pallas import tpu as pltpu
from jax.experimental.pallas import tpu_sc as plsc

N = 10000
D = 128
G = 256
E = 320000

NC = 2
NS = 16
NW = NC * NS

C = 100
NCHUNK = 100
HALF = 25
NBUF = 3
PER_TILE = NCHUNK * C
N_PAD = 10240
ZROWS = 128
ROWS_PER_TILE = N_PAD // NS

_PREC = jax.lax.Precision.DEFAULT


def _sc_segment_partials(feats, src3, dst3):
  mesh = plsc.VectorSubcoreMesh(core_axis_name="c", subcore_axis_name="s")

  @functools.partial(
      pl.kernel,
      out_type=jax.ShapeDtypeStruct((NC, N_PAD, D), jnp.float32),
      mesh=mesh,
      scratch_types=[
          pltpu.VMEM((HALF, C), jnp.int32),
          pltpu.VMEM((HALF, C), jnp.int32),
          pltpu.VMEM((C, D), jnp.float32),
          pltpu.VMEM((C, D), jnp.float32),
          pltpu.VMEM((C, D), jnp.float32),
          pltpu.VMEM_SHARED((N_PAD, D), jnp.float32),
          pltpu.SemaphoreType.DMA,
          pltpu.SemaphoreType.DMA,
          pltpu.SemaphoreType.DMA,
      ],
  )
  def k(feats_hbm, src_hbm, dst_hbm, out_hbm, sidx, didx, rows0, rows1, rows2,
        accum, sem0, sem1, sem2):
    cid = lax.axis_index("c")
    sid = lax.axis_index("s")
    wid = cid * NS + sid

    def zfill(r, carry):
      for j in range(D // 16):
        rows0[r, pl.ds(j * 16, 16)] = jnp.zeros((16,), jnp.float32)
      return carry

    lax.fori_loop(0, C, zfill, 0)

    def zcopy(j, carry):
      pltpu.sync_copy(rows0, accum.at[pl.ds(sid * ROWS_PER_TILE + j * C, C)])
      return carry

    lax.fori_loop(0, ROWS_PER_TILE // C, zcopy, 0)
    pltpu.sync_copy(
        rows0.at[pl.ds(0, ROWS_PER_TILE % C)],
        accum.at[pl.ds(sid * ROWS_PER_TILE + (ROWS_PER_TILE // C) * C,
                       ROWS_PER_TILE % C)])
    plsc.subcore_barrier()

    bufs = [(rows0, sem0), (rows1, sem1), (rows2, sem2)]

    for h in range(NCHUNK // HALF):
      pltpu.sync_copy(src_hbm.at[wid, h], sidx)
      pltpu.sync_copy(dst_hbm.at[wid, h], didx)
      for b in range(NBUF):
        pltpu.async_copy(feats_hbm.at[sidx.at[b]], bufs[b][0], bufs[b][1])

      def chunk(i, carry):
        j = NBUF * i
        for b in range(NBUF):
          rws, sm = bufs[b]
          pltpu.make_async_copy(feats_hbm.at[sidx.at[j + b]], rws, sm).wait()
          pltpu.sync_copy(rws, accum.at[didx.at[j + b]], add=True)

          @pl.when(j + b + NBUF < HALF)
          def _():
            pltpu.async_copy(feats_hbm.at[sidx.at[j + b + NBUF]], rws, sm)

        return carry

      lax.fori_loop(0, HALF // NBUF, chunk, 0)
      pltpu.make_async_copy(feats_hbm.at[sidx.at[HALF - 1]], bufs[0][0],
                            bufs[0][1]).wait()
      pltpu.sync_copy(bufs[0][0], accum.at[didx.at[HALF - 1]], add=True)
    plsc.subcore_barrier()

    pltpu.sync_copy(
        accum.at[pl.ds(sid * ROWS_PER_TILE, ROWS_PER_TILE)],
        out_hbm.at[cid, pl.ds(sid * ROWS_PER_TILE, ROWS_PER_TILE)],
    )

  return k(feats, src3, dst3)


def _tc_mm_body(x_ref, w_ref, b_ref, o_ref):
  o_ref[...] = lax.dot_general(x_ref[...], w_ref[...], (((1,), (0,)), ((), ())),
                               precision=_PREC,
                               preferred_element_type=jnp.float32) + b_ref[...]


def _tc_mm(x, w, b):
  R = 2000
  grid = N // R
  return pl.pallas_call(
      _tc_mm_body,
      grid=(grid,),
      in_specs=[
          pl.BlockSpec((R, D), lambda i: (i, 0)),
          pl.BlockSpec((D, D), lambda i: (0, 0)),
          pl.BlockSpec((1, D), lambda i: (0, 0)),
      ],
      out_specs=pl.BlockSpec((R, D), lambda i: (i, 0)),
      out_shape=jax.ShapeDtypeStruct((N, D), jnp.float32),
  )(x, w, b)


def _tc_layer_body(p_ref, t_ref, wr_ref, wt2_ref, b2_ref, h_ref, t2_ref):
  agg = p_ref[0] + p_ref[1]
  h = lax.dot_general(agg, wr_ref[...], (((1,), (0,)), ((), ())),
                      precision=_PREC, preferred_element_type=jnp.float32)
  h = jnp.maximum(h + t_ref[...], 0.0)
  h_ref[...] = h
  t2_ref[...] = lax.dot_general(h, wt2_ref[...], (((1,), (0,)), ((), ())),
                                precision=_PREC,
                                preferred_element_type=jnp.float32) + b2_ref[...]


def _tc_layer(p, t, w_rel, w_root2, b2):
  R = 2000
  grid = N // R
  return pl.pallas_call(
      _tc_layer_body,
      grid=(grid,),
      in_specs=[
          pl.BlockSpec((NC, R, D), lambda i: (0, i, 0)),
          pl.BlockSpec((R, D), lambda i: (i, 0)),
          pl.BlockSpec((D, D), lambda i: (0, 0)),
          pl.BlockSpec((D, D), lambda i: (0, 0)),
          pl.BlockSpec((1, D), lambda i: (0, 0)),
      ],
      out_specs=[
          pl.BlockSpec((R, D), lambda i: (i, 0)),
          pl.BlockSpec((R, D), lambda i: (i, 0)),
      ],
      out_shape=[
          jax.ShapeDtypeStruct((N, D), jnp.float32),
          jax.ShapeDtypeStruct((N, D), jnp.float32),
      ],
  )(p, t, w_rel, w_root2, b2)


R2 = 2000
GRID2 = N // R2


def _tc_layer_pool_body(q_ref, t_ref, wr_ref, batch_ref,
                        out_ref, acc_ref, cnt_ref):
  step = pl.program_id(0)

  @pl.when(step == 0)
  def _():
    acc_ref[...] = jnp.zeros_like(acc_ref)
    cnt_ref[...] = jnp.zeros_like(cnt_ref)

  agg = q_ref[0] + q_ref[1]
  h2 = lax.dot_general(agg, wr_ref[...], (((1,), (0,)), ((), ())),
                       precision=_PREC, preferred_element_type=jnp.float32)
  h2 = jnp.maximum(h2 + t_ref[...], 0.0)

  bb = batch_ref[0, 0, :]
  iota_g = lax.broadcasted_iota(jnp.int32, (G, R2), 0)
  onehot = (iota_g == bb[None, :]).astype(jnp.float32)
  acc_ref[...] += lax.dot_general(onehot, h2, (((1,), (0,)), ((), ())),
                                  precision=_PREC,
                                  preferred_element_type=jnp.float32)
  cnt_ref[...] += jnp.broadcast_to(
      jnp.sum(onehot, axis=1, keepdims=True), (G, D))

  @pl.when(step == GRID2 - 1)
  def _():
    out_ref[...] = acc_ref[...] / jnp.maximum(cnt_ref[...], 1.0)


def _tc_layer_pool(q, t, w_rel, batch3d):
  return pl.pallas_call(
      _tc_layer_pool_body,
      grid=(GRID2,),
      in_specs=[
          pl.BlockSpec((NC, R2, D), lambda i: (0, i, 0)),
          pl.BlockSpec((R2, D), lambda i: (i, 0)),
          pl.BlockSpec((D, D), lambda i: (0, 0)),
          pl.BlockSpec((1, 1, R2), lambda i: (i, 0, 0)),
      ],
      out_specs=pl.BlockSpec((G, D), lambda i: (0, 0)),
      out_shape=jax.ShapeDtypeStruct((G, D), jnp.float32),
      scratch_shapes=[
          pltpu.VMEM((G, D), jnp.float32),
          pltpu.VMEM((G, D), jnp.float32),
      ],
  )(q, t, w_rel, batch3d)


def kernel(x, edge_index, batch, W_rel1, b_rel1, W_root1, W_rel2, b_rel2, W_root2):
  src3 = edge_index[0].reshape(NW, NCHUNK // HALF, HALF, C)
  dst3 = edge_index[1].reshape(NW, NCHUNK // HALF, HALF, C)

  b1 = b_rel1.reshape(1, D)
  b2 = b_rel2.reshape(1, D)
  batch3d = batch.reshape(GRID2, 1, R2)

  t1 = _tc_mm(x, W_root1, b1)
  p = _sc_segment_partials(x, src3, dst3)
  h, t2 = _tc_layer(p, t1, W_rel1, W_root2, b2)
  q = _sc_segment_partials(h, src3, dst3)
  out = _tc_layer_pool(q, t2, W_rel2, batch3d)
  return out


# speedup vs baseline: 1.0085x; 1.0085x over previous
import functools

import jax
import jax.numpy as jnp
from jax import lax
from jax.experimental import pallas as pl
from jax.experimental.pallas import tpu as pltpu
from jax.experimental.pallas import tpu_sc as plsc

N = 10000
D = 128
G = 256
E = 320000

NC = 2
NS = 16
NW = NC * NS

C = 100
NCHUNK = 100
HALF = 25
NBUF = 3
PER_TILE = NCHUNK * C
N_PAD = 10240
ZROWS = 128
ROWS_PER_TILE = N_PAD // NS

_PREC = jax.lax.Precision.DEFAULT


def _sc_segment_partials(feats, src3, dst3):
  mesh = plsc.VectorSubcoreMesh(core_axis_name="c", subcore_axis_name="s")

  @functools.partial(
      pl.kernel,
      out_type=jax.ShapeDtypeStruct((NC, N_PAD, D), jnp.float32),
      mesh=mesh,
      scratch_types=[
          pltpu.VMEM((HALF, C), jnp.int32),
          pltpu.VMEM((HALF, C), jnp.int32),
          pltpu.VMEM((C, D), jnp.float32),
          pltpu.VMEM((C, D), jnp.float32),
          pltpu.VMEM((C, D), jnp.float32),
          pltpu.VMEM_SHARED((N_PAD, D), jnp.float32),
          pltpu.SemaphoreType.DMA,
          pltpu.SemaphoreType.DMA,
          pltpu.SemaphoreType.DMA,
      ],
  )
  def k(feats_hbm, src_hbm, dst_hbm, out_hbm, sidx, didx, rows0, rows1, rows2,
        accum, sem0, sem1, sem2):
    cid = lax.axis_index("c")
    sid = lax.axis_index("s")
    wid = cid * NS + sid

    def zfill(r, carry):
      for j in range(D // 16):
        rows0[r, pl.ds(j * 16, 16)] = jnp.zeros((16,), jnp.float32)
      return carry

    lax.fori_loop(0, C, zfill, 0)

    def zcopy(j, carry):
      pltpu.sync_copy(rows0, accum.at[pl.ds(sid * ROWS_PER_TILE + j * C, C)])
      return carry

    lax.fori_loop(0, ROWS_PER_TILE // C, zcopy, 0)
    pltpu.sync_copy(
        rows0.at[pl.ds(0, ROWS_PER_TILE % C)],
        accum.at[pl.ds(sid * ROWS_PER_TILE + (ROWS_PER_TILE // C) * C,
                       ROWS_PER_TILE % C)])
    plsc.subcore_barrier()

    bufs = [(rows0, sem0), (rows1, sem1), (rows2, sem2)]

    for h in range(NCHUNK // HALF):
      pltpu.sync_copy(src_hbm.at[wid, h], sidx)
      pltpu.sync_copy(dst_hbm.at[wid, h], didx)
      for b in range(NBUF):
        pltpu.async_copy(feats_hbm.at[sidx.at[b]], bufs[b][0], bufs[b][1])

      def chunk(i, carry):
        j = NBUF * i
        for b in range(NBUF):
          rws, sm = bufs[b]
          pltpu.make_async_copy(feats_hbm.at[sidx.at[j + b]], rws, sm).wait()
          pltpu.sync_copy(rws, accum.at[didx.at[j + b]], add=True)

          @pl.when(j + b + NBUF < HALF)
          def _():
            pltpu.async_copy(feats_hbm.at[sidx.at[j + b + NBUF]], rws, sm)

        return carry

      lax.fori_loop(0, HALF // NBUF, chunk, 0)
      pltpu.make_async_copy(feats_hbm.at[sidx.at[HALF - 1]], bufs[0][0],
                            bufs[0][1]).wait()
      pltpu.sync_copy(bufs[0][0], accum.at[didx.at[HALF - 1]], add=True)
    plsc.subcore_barrier()

    pltpu.sync_copy(
        accum.at[pl.ds(sid * ROWS_PER_TILE, ROWS_PER_TILE)],
        out_hbm.at[cid, pl.ds(sid * ROWS_PER_TILE, ROWS_PER_TILE)],
    )

  return k(feats, src3, dst3)


def _tc_mm_body(x_ref, w_ref, b_ref, o_ref):
  o_ref[...] = lax.dot_general(x_ref[...], w_ref[...], (((1,), (0,)), ((), ())),
                               precision=_PREC,
                               preferred_element_type=jnp.float32) + b_ref[...]


def _tc_mm(x, w, b):
  R = 2000
  grid = N // R
  return pl.pallas_call(
      _tc_mm_body,
      grid=(grid,),
      in_specs=[
          pl.BlockSpec((R, D), lambda i: (i, 0)),
          pl.BlockSpec((D, D), lambda i: (0, 0)),
          pl.BlockSpec((1, D), lambda i: (0, 0)),
      ],
      out_specs=pl.BlockSpec((R, D), lambda i: (i, 0)),
      out_shape=jax.ShapeDtypeStruct((N, D), jnp.float32),
  )(x, w, b)


def _tc_layer_body(p_ref, t_ref, wr_ref, h_ref):
  agg = p_ref[0] + p_ref[1]
  h = lax.dot_general(agg, wr_ref[...], (((1,), (0,)), ((), ())),
                      precision=_PREC, preferred_element_type=jnp.float32)
  h_ref[...] = jnp.maximum(h + t_ref[...], 0.0)


def _tc_layer(p, t, w_rel):
  R = 2000
  grid = N // R
  return pl.pallas_call(
      _tc_layer_body,
      grid=(grid,),
      in_specs=[
          pl.BlockSpec((NC, R, D), lambda i: (0, i, 0)),
          pl.BlockSpec((R, D), lambda i: (i, 0)),
          pl.BlockSpec((D, D), lambda i: (0, 0)),
      ],
      out_specs=pl.BlockSpec((R, D), lambda i: (i, 0)),
      out_shape=jax.ShapeDtypeStruct((N, D), jnp.float32),
  )(p, t, w_rel)


R2 = 2000
GRID2 = N // R2


def _tc_layer_pool_body(q_ref, t_ref, wr_ref, batch_ref,
                        out_ref, acc_ref, cnt_ref):
  step = pl.program_id(0)

  @pl.when(step == 0)
  def _():
    acc_ref[...] = jnp.zeros_like(acc_ref)
    cnt_ref[...] = jnp.zeros_like(cnt_ref)

  agg = q_ref[0] + q_ref[1]
  h2 = lax.dot_general(agg, wr_ref[...], (((1,), (0,)), ((), ())),
                       precision=_PREC, preferred_element_type=jnp.float32)
  h2 = jnp.maximum(h2 + t_ref[...], 0.0)

  bb = batch_ref[0, 0, :]
  iota_g = lax.broadcasted_iota(jnp.int32, (G, R2), 0)
  onehot = (iota_g == bb[None, :]).astype(jnp.float32)
  acc_ref[...] += lax.dot_general(onehot, h2, (((1,), (0,)), ((), ())),
                                  precision=_PREC,
                                  preferred_element_type=jnp.float32)
  cnt_ref[...] += jnp.broadcast_to(
      jnp.sum(onehot, axis=1, keepdims=True), (G, D))

  @pl.when(step == GRID2 - 1)
  def _():
    out_ref[...] = acc_ref[...] / jnp.maximum(cnt_ref[...], 1.0)


def _tc_layer_pool(q, t, w_rel, batch3d):
  return pl.pallas_call(
      _tc_layer_pool_body,
      grid=(GRID2,),
      in_specs=[
          pl.BlockSpec((NC, R2, D), lambda i: (0, i, 0)),
          pl.BlockSpec((R2, D), lambda i: (i, 0)),
          pl.BlockSpec((D, D), lambda i: (0, 0)),
          pl.BlockSpec((1, 1, R2), lambda i: (i, 0, 0)),
      ],
      out_specs=pl.BlockSpec((G, D), lambda i: (0, 0)),
      out_shape=jax.ShapeDtypeStruct((G, D), jnp.float32),
      scratch_shapes=[
          pltpu.VMEM((G, D), jnp.float32),
          pltpu.VMEM((G, D), jnp.float32),
      ],
  )(q, t, w_rel, batch3d)


def kernel(x, edge_index, batch, W_rel1, b_rel1, W_root1, W_rel2, b_rel2, W_root2):
  src3 = edge_index[0].reshape(NW, NCHUNK // HALF, HALF, C)
  dst3 = edge_index[1].reshape(NW, NCHUNK // HALF, HALF, C)

  b1 = b_rel1.reshape(1, D)
  b2 = b_rel2.reshape(1, D)
  batch3d = batch.reshape(GRID2, 1, R2)

  t1 = _tc_mm(x, W_root1, b1)
  p = _sc_segment_partials(x, src3, dst3)
  h = _tc_layer(p, t1, W_rel1)
  t2 = _tc_mm(h, W_root2, b2)
  q = _sc_segment_partials(h, src3, dst3)
  out = _tc_layer_pool(q, t2, W_rel2, batch3d)
  return out


# speedup vs baseline: 1.0087x; 1.0002x over previous
import functools

import jax
import jax.numpy as jnp
from jax import lax
from jax.experimental import pallas as pl
from jax.experimental.pallas import tpu as pltpu
from jax.experimental.pallas import tpu_sc as plsc

N = 10000
D = 128
G = 256
E = 320000

NC = 2
NS = 16
NW = NC * NS

C = 100
NCHUNK = 100
HALF = 25
NBUF = 3
PER_TILE = NCHUNK * C
N_PAD = 10240
ROWS_PER_TILE = N_PAD // NS

_PREC = jax.lax.Precision.DEFAULT


def _sc_segment_partials(feats, src3, dst3):
  mesh = plsc.VectorSubcoreMesh(core_axis_name="c", subcore_axis_name="s")

  @functools.partial(
      pl.kernel,
      out_type=jax.ShapeDtypeStruct((NC, N_PAD, D), jnp.float32),
      mesh=mesh,
      scratch_types=[
          pltpu.VMEM((HALF, C), jnp.int32),
          pltpu.VMEM((HALF, C), jnp.int32),
          pltpu.VMEM((C, D), jnp.float32),
          pltpu.VMEM((C, D), jnp.float32),
          pltpu.VMEM((C, D), jnp.float32),
          pltpu.VMEM_SHARED((N_PAD, D), jnp.float32),
          pltpu.SemaphoreType.DMA,
          pltpu.SemaphoreType.DMA,
          pltpu.SemaphoreType.DMA,
      ],
  )
  def k(feats_hbm, src_hbm, dst_hbm, out_hbm, sidx, didx, rows0, rows1, rows2,
        accum, sem0, sem1, sem2):
    cid = lax.axis_index("c")
    sid = lax.axis_index("s")
    wid = cid * NS + sid

    def zfill(r, carry):
      for j in range(D // 16):
        rows0[r, pl.ds(j * 16, 16)] = jnp.zeros((16,), jnp.float32)
      return carry

    lax.fori_loop(0, C, zfill, 0)

    def zcopy(j, carry):
      pltpu.sync_copy(rows0, accum.at[pl.ds(sid * ROWS_PER_TILE + j * C, C)])
      return carry

    lax.fori_loop(0, ROWS_PER_TILE // C, zcopy, 0)
    pltpu.sync_copy(
        rows0.at[pl.ds(0, ROWS_PER_TILE % C)],
        accum.at[pl.ds(sid * ROWS_PER_TILE + (ROWS_PER_TILE // C) * C,
                       ROWS_PER_TILE % C)])
    plsc.subcore_barrier()

    bufs = [(rows0, sem0), (rows1, sem1), (rows2, sem2)]

    for h in range(NCHUNK // HALF):
      pltpu.sync_copy(src_hbm.at[wid, h], sidx)
      pltpu.sync_copy(dst_hbm.at[wid, h], didx)
      for b in range(NBUF):
        pltpu.async_copy(feats_hbm.at[sidx.at[b]], bufs[b][0], bufs[b][1])

      def chunk(i, carry):
        j = NBUF * i
        for b in range(NBUF):
          rws, sm = bufs[b]
          pltpu.make_async_copy(feats_hbm.at[sidx.at[j + b]], rws, sm).wait()
          pltpu.sync_copy(rws, accum.at[didx.at[j + b]], add=True)

          @pl.when(j + b + NBUF < HALF)
          def _():
            pltpu.async_copy(feats_hbm.at[sidx.at[j + b + NBUF]], rws, sm)

        return carry

      lax.fori_loop(0, HALF // NBUF, chunk, 0)
      pltpu.make_async_copy(feats_hbm.at[sidx.at[HALF - 1]], bufs[0][0],
                            bufs[0][1]).wait()
      pltpu.sync_copy(bufs[0][0], accum.at[didx.at[HALF - 1]], add=True)
    plsc.subcore_barrier()

    pltpu.sync_copy(
        accum.at[pl.ds(sid * ROWS_PER_TILE, ROWS_PER_TILE)],
        out_hbm.at[cid, pl.ds(sid * ROWS_PER_TILE, ROWS_PER_TILE)],
    )

  return k(feats, src3, dst3)


def _tc_mm_body(x_ref, w_ref, b_ref, o_ref):
  o_ref[...] = lax.dot_general(x_ref[...], w_ref[...], (((1,), (0,)), ((), ())),
                               precision=_PREC,
                               preferred_element_type=jnp.float32) + b_ref[...]


def _tc_mm(x, w, b):
  R = 2000
  grid = N // R
  return pl.pallas_call(
      _tc_mm_body,
      grid=(grid,),
      in_specs=[
          pl.BlockSpec((R, D), lambda i: (i, 0)),
          pl.BlockSpec((D, D), lambda i: (0, 0)),
          pl.BlockSpec((1, D), lambda i: (0, 0)),
      ],
      out_specs=pl.BlockSpec((R, D), lambda i: (i, 0)),
      out_shape=jax.ShapeDtypeStruct((N, D), jnp.float32),
  )(x, w, b)


def _tc_layer_body(p_ref, t_ref, wr_ref, h_ref):
  agg = p_ref[0] + p_ref[1]
  h = lax.dot_general(agg, wr_ref[...], (((1,), (0,)), ((), ())),
                      precision=_PREC, preferred_element_type=jnp.float32)
  h_ref[...] = jnp.maximum(h + t_ref[...], 0.0)


def _tc_layer(p, t, w_rel):
  R = 2000
  grid = N // R
  return pl.pallas_call(
      _tc_layer_body,
      grid=(grid,),
      in_specs=[
          pl.BlockSpec((NC, R, D), lambda i: (0, i, 0)),
          pl.BlockSpec((R, D), lambda i: (i, 0)),
          pl.BlockSpec((D, D), lambda i: (0, 0)),
      ],
      out_specs=pl.BlockSpec((R, D), lambda i: (i, 0)),
      out_shape=jax.ShapeDtypeStruct((N, D), jnp.float32),
  )(p, t, w_rel)


R2 = 2000
GRID2 = N // R2


def _tc_layer_pool_body(q_ref, t_ref, wr_ref, batch_ref,
                        out_ref, acc_ref, cnt_ref):
  step = pl.program_id(0)

  @pl.when(step == 0)
  def _():
    acc_ref[...] = jnp.zeros_like(acc_ref)
    cnt_ref[...] = jnp.zeros_like(cnt_ref)

  agg = q_ref[0] + q_ref[1]
  h2 = lax.dot_general(agg, wr_ref[...], (((1,), (0,)), ((), ())),
                       precision=_PREC, preferred_element_type=jnp.float32)
  h2 = jnp.maximum(h2 + t_ref[...], 0.0)

  bb = batch_ref[0, 0, :]
  iota_g = lax.broadcasted_iota(jnp.int32, (G, R2), 0)
  onehot = (iota_g == bb[None, :]).astype(jnp.float32)
  acc_ref[...] += lax.dot_general(onehot, h2, (((1,), (0,)), ((), ())),
                                  precision=_PREC,
                                  preferred_element_type=jnp.float32)
  cnt_ref[...] += jnp.broadcast_to(
      jnp.sum(onehot, axis=1, keepdims=True), (G, D))

  @pl.when(step == GRID2 - 1)
  def _():
    out_ref[...] = acc_ref[...] / jnp.maximum(cnt_ref[...], 1.0)


def _tc_layer_pool(q, t, w_rel, batch3d):
  return pl.pallas_call(
      _tc_layer_pool_body,
      grid=(GRID2,),
      in_specs=[
          pl.BlockSpec((NC, R2, D), lambda i: (0, i, 0)),
          pl.BlockSpec((R2, D), lambda i: (i, 0)),
          pl.BlockSpec((D, D), lambda i: (0, 0)),
          pl.BlockSpec((1, 1, R2), lambda i: (i, 0, 0)),
      ],
      out_specs=pl.BlockSpec((G, D), lambda i: (0, 0)),
      out_shape=jax.ShapeDtypeStruct((G, D), jnp.float32),
      scratch_shapes=[
          pltpu.VMEM((G, D), jnp.float32),
          pltpu.VMEM((G, D), jnp.float32),
      ],
  )(q, t, w_rel, batch3d)


def kernel(x, edge_index, batch, W_rel1, b_rel1, W_root1, W_rel2, b_rel2, W_root2):
  src3 = edge_index[0].reshape(NW, NCHUNK // HALF, HALF, C)
  dst3 = edge_index[1].reshape(NW, NCHUNK // HALF, HALF, C)

  b1 = b_rel1.reshape(1, D)
  b2 = b_rel2.reshape(1, D)
  batch3d = batch.reshape(GRID2, 1, R2)

  t1 = _tc_mm(x, W_root1, b1)
  p = _sc_segment_partials(x, src3, dst3)
  h = _tc_layer(p, t1, W_rel1)
  t2 = _tc_mm(h, W_root2, b2)
  q = _sc_segment_partials(h, src3, dst3)
  out = _tc_layer_pool(q, t2, W_rel2, batch3d)
  return out
